# baseline (speedup 1.0000x reference)
import functools

import jax
import jax.numpy as jnp
from jax import lax
from jax.experimental import pallas as pl
from jax.experimental.pallas import tpu as pltpu
from jax.experimental.pallas import tpu_sc as plsc

_B, _P, _D = 64, 1024, 768
_NC, _NS, _L = 2, 16, 16
_NW = _NC * _NS
_PW = _P // _NW
_W = _PW * _D
_SLOTS = 3
_UN = 8


def _add_table(buf, tab):
    def body(j, carry):
        off = j * (_L * _UN)
        for k in range(_UN):
            o = off + k * _L
            buf[pl.ds(o, _L)] = buf[pl.ds(o, _L)] + tab[pl.ds(o, _L)]
        return carry

    lax.fori_loop(0, _W // (_L * _UN), body, 0)


def _make_sc_kernel():
    mesh = plsc.VectorSubcoreMesh(core_axis_name="c", subcore_axis_name="s")

    @functools.partial(
        pl.kernel,
        mesh=mesh,
        out_type=jax.ShapeDtypeStruct((_B, _P * _D), jnp.float32),
        scratch_types=(
            [pltpu.VMEM((_W,), jnp.float32)]
            + [pltpu.VMEM((_W,), jnp.float32)] * _SLOTS
            + [pltpu.SemaphoreType.DMA] * _SLOTS
            + [pltpu.SemaphoreType.DMA] * _SLOTS
        ),
    )
    def sc_kernel(x_hbm, t_hbm, out_hbm, tab, *rest):
        bufs = rest[:_SLOTS]
        sin = rest[_SLOTS:2 * _SLOTS]
        sout = rest[2 * _SLOTS:3 * _SLOTS]
        wid = lax.axis_index("s") * _NC + lax.axis_index("c")
        base = wid * _W

        pltpu.sync_copy(t_hbm.at[pl.ds(base, _W)], tab)

        h_in = [None] * _SLOTS
        h_out = [None] * _SLOTS
        for b in range(min(_SLOTS, _B)):
            h_in[b] = pltpu.async_copy(
                x_hbm.at[b, pl.ds(base, _W)], bufs[b], sin[b])
        for b in range(_B):
            s = b % _SLOTS
            h_in[s].wait()
            _add_table(bufs[s], tab)
            h_out[s] = pltpu.async_copy(
                bufs[s], out_hbm.at[b, pl.ds(base, _W)], sout[s])
            nb = b + _SLOTS
            if nb < _B:
                h_out[s].wait()
                h_in[s] = pltpu.async_copy(
                    x_hbm.at[nb, pl.ds(base, _W)], bufs[s], sin[s])
        for s in range(_SLOTS):
            if h_out[s] is not None:
                h_out[s].wait()

    return sc_kernel


_sc_kernel = _make_sc_kernel()


def kernel(x, table):
    out = _sc_kernel(x.reshape(_B, _P * _D), table.reshape(_P * _D))
    return out.reshape(_B, _P, _D)


# speedup vs baseline: 1.0676x; 1.0676x over previous
import functools

import jax
import jax.numpy as jnp
from jax import lax
from jax.experimental import pallas as pl
from jax.experimental.pallas import tpu as pltpu
from jax.experimental.pallas import tpu_sc as plsc

_B, _P, _D = 64, 1024, 768
_NC, _NS, _L = 2, 16, 16
_NW = _NC * _NS
_PW = _P // _NW
_W = _PW * _D
_SLOTS = 4
_LEAD = 2
_UN = 16


def _add_table(buf, tab):
    def body(j, carry):
        off = j * (_L * _UN)
        for k in range(_UN):
            o = off + k * _L
            buf[pl.ds(o, _L)] = buf[pl.ds(o, _L)] + tab[pl.ds(o, _L)]
        return carry

    lax.fori_loop(0, _W // (_L * _UN), body, 0)


def _make_sc_kernel():
    mesh = plsc.VectorSubcoreMesh(core_axis_name="c", subcore_axis_name="s")

    @functools.partial(
        pl.kernel,
        mesh=mesh,
        out_type=jax.ShapeDtypeStruct((_B, _P * _D), jnp.float32),
        scratch_types=(
            [pltpu.VMEM((_W,), jnp.float32)]
            + [pltpu.VMEM((_W,), jnp.float32)] * _SLOTS
            + [pltpu.SemaphoreType.DMA] * _SLOTS
            + [pltpu.SemaphoreType.DMA] * _SLOTS
        ),
    )
    def sc_kernel(x_hbm, t_hbm, out_hbm, tab, *rest):
        bufs = rest[:_SLOTS]
        sin = rest[_SLOTS:2 * _SLOTS]
        sout = rest[2 * _SLOTS:3 * _SLOTS]
        wid = lax.axis_index("s") * _NC + lax.axis_index("c")
        base = wid * _W

        pltpu.sync_copy(t_hbm.at[pl.ds(base, _W)], tab)

        h_in = [None] * _SLOTS
        h_out = [None] * _SLOTS
        for b in range(min(_LEAD, _B)):
            h_in[b] = pltpu.async_copy(
                x_hbm.at[b, pl.ds(base, _W)], bufs[b], sin[b])
        for b in range(_B):
            s = b % _SLOTS
            nb = b + _LEAD
            if nb < _B:
                so = nb % _SLOTS
                if h_out[so] is not None:
                    h_out[so].wait()
                h_in[so] = pltpu.async_copy(
                    x_hbm.at[nb, pl.ds(base, _W)], bufs[so], sin[so])
            h_in[s].wait()
            _add_table(bufs[s], tab)
            h_out[s] = pltpu.async_copy(
                bufs[s], out_hbm.at[b, pl.ds(base, _W)], sout[s])
        for s in range(_SLOTS):
            if h_out[s] is not None:
                h_out[s].wait()

    return sc_kernel


_sc_kernel = _make_sc_kernel()


def kernel(x, table):
    out = _sc_kernel(x.reshape(_B, _P * _D), table.reshape(_P * _D))
    return out.reshape(_B, _P, _D)


# speedup vs baseline: 2.9409x; 2.7546x over previous
import functools

import jax
import jax.numpy as jnp
from jax import lax
from jax.experimental import pallas as pl
from jax.experimental.pallas import tpu as pltpu
from jax.experimental.pallas import tpu_sc as plsc

_B, _P, _D = 64, 1024, 768
_NC, _NS, _L = 2, 16, 16
_NW = _NC * _NS
_PW = _P // _NW
_W = _PW * _D
_SLOTS = 4
_LEAD = 2
_UN = 16


def _add_table(buf, tab):
    def row_body(r, carry):
        for k in range(_D // _L):
            c = k * _L
            buf[r, pl.ds(c, _L)] = buf[r, pl.ds(c, _L)] + tab[r, pl.ds(c, _L)]
        return carry

    lax.fori_loop(0, _PW, row_body, 0)


def _make_sc_kernel():
    mesh = plsc.VectorSubcoreMesh(core_axis_name="c", subcore_axis_name="s")

    @functools.partial(
        pl.kernel,
        mesh=mesh,
        out_type=jax.ShapeDtypeStruct((_B, _P, _D), jnp.float32),
        scratch_types=(
            [pltpu.VMEM((_PW, _D), jnp.float32)]
            + [pltpu.VMEM((_PW, _D), jnp.float32)] * _SLOTS
            + [pltpu.SemaphoreType.DMA] * _SLOTS
            + [pltpu.SemaphoreType.DMA] * _SLOTS
        ),
    )
    def sc_kernel(x_hbm, t_hbm, out_hbm, tab, *rest):
        bufs = rest[:_SLOTS]
        sin = rest[_SLOTS:2 * _SLOTS]
        sout = rest[2 * _SLOTS:3 * _SLOTS]
        wid = lax.axis_index("s") * _NC + lax.axis_index("c")
        base = wid * _PW

        pltpu.sync_copy(t_hbm.at[pl.ds(base, _PW), :], tab)

        def start_in(b, s):
            pltpu.async_copy(x_hbm.at[b, pl.ds(base, _PW), :], bufs[s], sin[s])

        def start_out(b, s):
            pltpu.async_copy(bufs[s], out_hbm.at[b, pl.ds(base, _PW), :],
                             sout[s])

        def wait_in(s):
            pltpu.make_async_copy(
                x_hbm.at[0, pl.ds(base, _PW), :], bufs[s], sin[s]).wait()

        def wait_out(s):
            pltpu.make_async_copy(
                bufs[s], out_hbm.at[0, pl.ds(base, _PW), :], sout[s]).wait()

        start_in(0, 0)
        start_in(1, 1)
        start_in(2, 2)
        wait_in(0)
        _add_table(bufs[0], tab)
        start_out(0, 0)
        start_in(3, 3)
        wait_in(1)
        _add_table(bufs[1], tab)
        start_out(1, 1)

        def group(g, carry):
            b0 = 2 + 4 * g
            for j in range(4):
                b = b0 + j
                s = (2 + j) % _SLOTS
                so = j
                wait_out(so)
                start_in(b + 2, so)
                wait_in(s)
                _add_table(bufs[s], tab)
                start_out(b, s)
            return carry

        lax.fori_loop(0, (_B - 4) // 4, group, 0)

        wait_in(2)
        _add_table(bufs[2], tab)
        start_out(_B - 2, 2)
        wait_in(3)
        _add_table(bufs[3], tab)
        start_out(_B - 1, 3)
        for s in range(_SLOTS):
            wait_out(s)

    return sc_kernel


_sc_kernel = _make_sc_kernel()


def kernel(x, table):
    return _sc_kernel(x, table)
